# Initial kernel scaffold; baseline (speedup 1.0000x reference)
import jax
import jax.numpy as jnp
from jax.experimental import pallas as pl


def kernel(x, edge_index_0, edge_index_1, node_graph_ids, eps, W_a, b_a, g, be, W_b, b_b, Wo1, bo1, Wo2, bo2, Wo3, bo3):
    raise NotImplementedError("write your pallas kernel here")


# speedup vs baseline: 1.7785x; 1.7785x over previous
import functools

import jax
import jax.numpy as jnp
from jax import lax
from jax.experimental import pallas as pl
from jax.experimental.pallas import tpu as pltpu
from jax.experimental.pallas import tpu_sc as plsc

N = 10000
D = 128
E = 320000
G = 128

NC = 2
NS = 16
NW = NC * NS
CHUNK = 128
EPW = 10240
NCHUNKS = EPW // CHUNK
E_PAD = NW * EPW
SH_ROWS = 10240
ZCOPIES = SH_ROWS // NS // CHUNK
OUT_PER_S = SH_ROWS // NS


def _pad_edges(ei):
    pad = E_PAD - E
    src_p = jnp.concatenate([ei[0], jnp.zeros((pad,), jnp.int32)])
    dst_p = jnp.concatenate([ei[1], jnp.full((pad,), N, jnp.int32)])
    return src_p, dst_p


def _sc_aggregate(h, src_p, dst_p):
    mesh = plsc.VectorSubcoreMesh(core_axis_name="c", subcore_axis_name="s")

    @functools.partial(
        pl.kernel,
        out_type=jax.ShapeDtypeStruct((NC, SH_ROWS, D), jnp.float32),
        mesh=mesh,
        scratch_types=[
            pltpu.VMEM((CHUNK,), jnp.int32),
            pltpu.VMEM((CHUNK,), jnp.int32),
            pltpu.VMEM((CHUNK, D), jnp.float32),
            pltpu.VMEM_SHARED((SH_ROWS, D), jnp.float32),
            pltpu.SemaphoreType.DMA,
        ],
    )
    def agg_kernel(h_hbm, src_hbm, dst_hbm, out_hbm, sidx, didx, rows, shared, sem):
        c = lax.axis_index("c")
        s = lax.axis_index("s")
        wid = c * NS + s

        def zrow(r, carry):
            def zcol(j, carry2):
                rows[r, pl.ds(j * 16, 16)] = jnp.zeros((16,), jnp.float32)
                return carry2
            return lax.fori_loop(0, D // 16, zcol, carry)
        lax.fori_loop(0, CHUNK, zrow, 0)

        zbase = s * (SH_ROWS // NS)
        def zcopy(k, carry):
            pltpu.sync_copy(rows, shared.at[pl.ds(zbase + k * CHUNK, CHUNK)])
            return carry
        lax.fori_loop(0, ZCOPIES, zcopy, 0)
        plsc.subcore_barrier()

        base0 = wid * EPW
        def chunk_body(i, carry):
            base = pl.multiple_of(base0 + i * CHUNK, CHUNK)
            pltpu.sync_copy(src_hbm.at[pl.ds(base, CHUNK)], sidx)
            pltpu.sync_copy(dst_hbm.at[pl.ds(base, CHUNK)], didx)
            pltpu.async_copy(h_hbm.at[sidx], rows, sem).wait()
            pltpu.sync_copy(rows, shared.at[didx], add=True)
            return carry
        lax.fori_loop(0, NCHUNKS, chunk_body, 0)
        plsc.subcore_barrier()

        obase = s * OUT_PER_S
        pltpu.sync_copy(shared.at[pl.ds(obase, OUT_PER_S)],
                        out_hbm.at[c, pl.ds(obase, OUT_PER_S)])

    return agg_kernel(h, src_p, dst_p)


def _dense_block(h, agg_pair, scale, Wa, ba, gm, bet, Wb, bb, act):
    def body(h_ref, a_ref, sc_ref, wa_ref, ba_ref, g_ref, be_ref, wb_ref,
             bb_ref, o_ref):
        z = sc_ref[...] * h_ref[...] + a_ref[0, :N] + a_ref[1, :N]
        z = jnp.dot(z, wa_ref[...], preferred_element_type=jnp.float32) + ba_ref[...]
        m = jnp.mean(z, axis=0, keepdims=True)
        v = jnp.mean(jnp.square(z - m), axis=0, keepdims=True)
        z = g_ref[...] * (z - m) / jnp.sqrt(v + 1e-5) + be_ref[...]
        z = jnp.maximum(z, 0.0)
        z = jnp.dot(z, wb_ref[...], preferred_element_type=jnp.float32) + bb_ref[...]
        if act:
            z = jnp.maximum(z, 0.0)
        o_ref[...] = z

    return pl.pallas_call(
        body,
        out_shape=jax.ShapeDtypeStruct((N, D), jnp.float32),
    )(h, agg_pair, scale, Wa, ba, gm, bet, Wb, bb)


def _readout(h0, h1, ids_row, Wo1r, bo1r, Wo2, bo2r, Wo3p, bo3r):
    def body(h0_ref, h1_ref, ids_ref, w1_ref, b1_ref, w2_ref, b2_ref, w3_ref,
             b3_ref, o_ref):
        gi = lax.broadcasted_iota(jnp.int32, (G, N), 0)
        S = jnp.where(gi == ids_ref[...], 1.0, 0.0)
        hg0 = jnp.dot(S, h0_ref[...], preferred_element_type=jnp.float32)
        hg1 = jnp.dot(S, h1_ref[...], preferred_element_type=jnp.float32)
        o = (jnp.dot(hg0, w1_ref[0], preferred_element_type=jnp.float32)
             + jnp.dot(hg1, w1_ref[1], preferred_element_type=jnp.float32)
             + b1_ref[...])
        o = jnp.maximum(o, 0.0)
        o = jnp.dot(o, w2_ref[...], preferred_element_type=jnp.float32) + b2_ref[...]
        o = jnp.maximum(o, 0.0)
        o = jnp.dot(o, w3_ref[...], preferred_element_type=jnp.float32) + b3_ref[...]
        o_ref[...] = o

    return pl.pallas_call(
        body,
        out_shape=jax.ShapeDtypeStruct((G, D), jnp.float32),
    )(h0, h1, ids_row, Wo1r, bo1r, Wo2, bo2r, Wo3p, bo3r)


def kernel(x, edge_index_0, edge_index_1, node_graph_ids, eps, W_a, b_a, g, be,
           W_b, b_b, Wo1, bo1, Wo2, bo2, Wo3, bo3):
    outs = []
    for i, ei in enumerate((edge_index_0, edge_index_1)):
        src_p, dst_p = _pad_edges(ei)
        h = x
        for j in range(2):
            li = 2 * i + j
            agg_pair = _sc_aggregate(h, src_p, dst_p)
            scale = (1.0 + eps[li]).reshape(1, 1)
            h = _dense_block(h, agg_pair, scale,
                             W_a[li], b_a[li].reshape(1, D), g[li].reshape(1, D),
                             be[li].reshape(1, D), W_b[li], b_b[li].reshape(1, D),
                             act=(j == 0))
        outs.append(h)

    ids_row = node_graph_ids.reshape(1, N)
    Wo1r = Wo1.reshape(2, D, D)
    Wo3p = jnp.pad(Wo3, ((0, 0), (0, D - 1)))
    bo3r = jnp.pad(bo3, (0, D - 1)).reshape(1, D)
    o = _readout(outs[0], outs[1], ids_row, Wo1r, bo1.reshape(1, D), Wo2,
                 bo2.reshape(1, D), Wo3p, bo3r)
    return o[:, :1]


# speedup vs baseline: 2.5953x; 1.4593x over previous
import functools

import jax
import jax.numpy as jnp
from jax import lax
from jax.experimental import pallas as pl
from jax.experimental.pallas import tpu as pltpu
from jax.experimental.pallas import tpu_sc as plsc

N = 10000
D = 128
E = 320000
G = 128

NC = 2
NS = 16
CHUNK = 128
EPW = 20480
NCHUNKS = EPW // CHUNK
E_PAD = NS * EPW
NBUF = 2
IBLK = 8
NIB = 3
NBLK = NCHUNKS // IBLK
SH_ROWS = 10240
ZCOPIES = SH_ROWS // NS // CHUNK
OUT_PER_S = SH_ROWS // NS

HIGH = jax.lax.Precision.HIGHEST


def _sc_agg_pair(h2, src3, dst3):
    mesh = plsc.VectorSubcoreMesh(core_axis_name="c", subcore_axis_name="s")

    @functools.partial(
        pl.kernel,
        out_type=jax.ShapeDtypeStruct((NC, SH_ROWS, D), jnp.float32),
        mesh=mesh,
        scratch_types=[
            pltpu.VMEM((NIB, IBLK, CHUNK), jnp.int32),
            pltpu.VMEM((NIB, IBLK, CHUNK), jnp.int32),
            pltpu.VMEM((NBUF, CHUNK, D), jnp.float32),
            pltpu.VMEM_SHARED((SH_ROWS, D), jnp.float32),
            [pltpu.SemaphoreType.DMA] * NBUF,
            [pltpu.SemaphoreType.DMA] * NBUF,
            pltpu.SemaphoreType.DMA,
        ],
    )
    def agg_kernel(h_hbm, src_hbm, dst_hbm, out_hbm, sblk, dblk, rows, shared,
                   sg, ss, si):
        c = lax.axis_index("c")
        s = lax.axis_index("s")

        r0 = rows.at[0]
        def zrow(r, carry):
            def zcol(j, carry2):
                r0[r, pl.ds(j * 16, 16)] = jnp.zeros((16,), jnp.float32)
                return carry2
            return lax.fori_loop(0, D // 16, zcol, carry)
        lax.fori_loop(0, CHUNK, zrow, 0)
        zbase = s * (SH_ROWS // NS)
        def zcopy(k, carry):
            pltpu.sync_copy(r0, shared.at[pl.ds(zbase + k * CHUNK, CHUNK)])
            return carry
        lax.fori_loop(0, ZCOPIES, zcopy, 0)
        plsc.subcore_barrier()

        def idx_issue(kb, p):
            pltpu.async_copy(src_hbm.at[c, s, pl.ds(kb * IBLK, IBLK)],
                             sblk.at[p], si)
            pltpu.async_copy(dst_hbm.at[c, s, pl.ds(kb * IBLK, IBLK)],
                             dblk.at[p], si)

        def idx_wait():
            pltpu.make_async_copy(src_hbm.at[c, s, pl.ds(0, IBLK)],
                                  sblk.at[0], si).wait()
            pltpu.make_async_copy(dst_hbm.at[c, s, pl.ds(0, IBLK)],
                                  dblk.at[0], si).wait()

        def g_issue(p, pos, b):
            pltpu.async_copy(h_hbm.at[sblk.at[p, pos]], rows.at[b], sg[b])

        def g_wait(b):
            pltpu.make_async_copy(h_hbm.at[sblk.at[0, 0]], rows.at[b],
                                  sg[b]).wait()

        def s_issue(p, pos, b):
            pltpu.async_copy(rows.at[b], shared.at[dblk.at[p, pos]], ss[b],
                             add=True)

        def s_wait(b):
            pltpu.make_async_copy(rows.at[b], shared.at[dblk.at[0, 0]],
                                  ss[b]).wait()

        idx_issue(0, 0)

        def block(kb, carry):
            p = lax.rem(kb, NIB)
            pm1 = lax.rem(kb + (NIB - 1), NIB)
            pnx = lax.rem(kb + 1, NIB)
            idx_wait()

            @pl.when(kb < NBLK - 1)
            def _():
                idx_issue(kb + 1, pnx)

            for pos in range(IBLK):
                b = pos % NBUF
                if pos >= NBUF:
                    s_wait(b)
                else:
                    @pl.when(kb >= 1)
                    def _():
                        s_wait(b)
                g_issue(p, pos, b)
                if pos >= 1:
                    g_wait(1 - b)
                    s_issue(p, pos - 1, 1 - b)
                else:
                    @pl.when(kb >= 1)
                    def _():
                        g_wait((IBLK - 1) % NBUF)
                        s_issue(pm1, IBLK - 1, (IBLK - 1) % NBUF)
            return carry
        lax.fori_loop(0, NBLK, block, 0)

        lastb = (IBLK - 1) % NBUF
        g_wait(lastb)
        s_issue((NBLK - 1) % NIB, IBLK - 1, lastb)
        for b in range(NBUF):
            s_wait(b)
        plsc.subcore_barrier()

        obase = s * OUT_PER_S
        pltpu.sync_copy(shared.at[pl.ds(obase, OUT_PER_S)],
                        out_hbm.at[c, pl.ds(obase, OUT_PER_S)])

    return agg_kernel(h2, src3, dst3)


def _dense_pair(h, agg, scales, Wa2, ba2, g2, be2, Wb2, bb2, act, shared_h):
    def body(h_ref, a_ref, sc_ref, wa_ref, ba_ref, g_ref, be_ref, wb_ref,
             bb_ref, o_ref):
        hb = h_ref[...] if shared_h else h_ref[0]
        z = sc_ref[0] * hb + a_ref[0, :N]
        z = jnp.dot(z, wa_ref[0], precision=HIGH,
                    preferred_element_type=jnp.float32) + ba_ref[0]
        m = jnp.mean(z, axis=0, keepdims=True)
        v = jnp.mean(jnp.square(z - m), axis=0, keepdims=True)
        z = g_ref[0] * (z - m) / jnp.sqrt(v + 1e-5) + be_ref[0]
        z = jnp.maximum(z, 0.0)
        z = jnp.dot(z, wb_ref[0], precision=HIGH,
                    preferred_element_type=jnp.float32) + bb_ref[0]
        if act:
            z = jnp.maximum(z, 0.0)
        o_ref[0] = z

    h_spec = (pl.BlockSpec((N, D), lambda b: (0, 0)) if shared_h
              else pl.BlockSpec((1, N, D), lambda b: (b, 0, 0)))
    return pl.pallas_call(
        body,
        grid=(2,),
        in_specs=[
            h_spec,
            pl.BlockSpec((1, SH_ROWS, D), lambda b: (b, 0, 0)),
            pl.BlockSpec((1, 1, 1), lambda b: (b, 0, 0)),
            pl.BlockSpec((1, D, D), lambda b: (b, 0, 0)),
            pl.BlockSpec((1, 1, D), lambda b: (b, 0, 0)),
            pl.BlockSpec((1, 1, D), lambda b: (b, 0, 0)),
            pl.BlockSpec((1, 1, D), lambda b: (b, 0, 0)),
            pl.BlockSpec((1, D, D), lambda b: (b, 0, 0)),
            pl.BlockSpec((1, 1, D), lambda b: (b, 0, 0)),
        ],
        out_specs=pl.BlockSpec((1, N, D), lambda b: (b, 0, 0)),
        out_shape=jax.ShapeDtypeStruct((2, N, D), jnp.float32),
    )(h, agg, scales, Wa2, ba2, g2, be2, Wb2, bb2)


def _readout(hpair, ids_row, Wo1r, bo1r, Wo2, bo2r, Wo3p, bo3r):
    def body(h_ref, ids_ref, w1_ref, b1_ref, w2_ref, b2_ref, w3_ref, b3_ref,
             o_ref):
        gi = lax.broadcasted_iota(jnp.int32, (G, N), 0)
        S = jnp.where(gi == ids_ref[...], 1.0, 0.0)
        hg0 = jnp.dot(S, h_ref[0], precision=HIGH,
                      preferred_element_type=jnp.float32)
        hg1 = jnp.dot(S, h_ref[1], precision=HIGH,
                      preferred_element_type=jnp.float32)
        o = (jnp.dot(hg0, w1_ref[0], precision=HIGH,
                     preferred_element_type=jnp.float32)
             + jnp.dot(hg1, w1_ref[1], precision=HIGH,
                       preferred_element_type=jnp.float32)
             + b1_ref[...])
        o = jnp.maximum(o, 0.0)
        o = jnp.dot(o, w2_ref[...], precision=HIGH,
                    preferred_element_type=jnp.float32) + b2_ref[...]
        o = jnp.maximum(o, 0.0)
        o = jnp.dot(o, w3_ref[...], precision=HIGH,
                    preferred_element_type=jnp.float32) + b3_ref[...]
        o_ref[...] = o

    return pl.pallas_call(
        body,
        out_shape=jax.ShapeDtypeStruct((G, D), jnp.float32),
    )(hpair, ids_row, Wo1r, bo1r, Wo2, bo2r, Wo3p, bo3r)


def kernel(x, edge_index_0, edge_index_1, node_graph_ids, eps, W_a, b_a, g, be,
           W_b, b_b, Wo1, bo1, Wo2, bo2, Wo3, bo3):
    pad = E_PAD - E

    def prep(a, off, fill):
        return jnp.concatenate(
            [a + off, jnp.full((pad,), fill, jnp.int32)]
        ).reshape(NS, NCHUNKS, CHUNK)

    src_s1 = jnp.stack([prep(edge_index_0[0], 0, 0), prep(edge_index_1[0], 0, 0)])
    src_s2 = jnp.stack([prep(edge_index_0[0], 0, 0), prep(edge_index_1[0], N, 0)])
    dst3 = jnp.stack([prep(edge_index_0[1], 0, N), prep(edge_index_1[1], 0, N)])

    def wsel(w, i0, i1):
        return jnp.stack([w[i0], w[i1]])

    def bsel(w, i0, i1):
        return jnp.stack([w[i0], w[i1]]).reshape(2, 1, D)

    agg1 = _sc_agg_pair(x, src_s1, dst3)
    sc1 = jnp.stack([1.0 + eps[0], 1.0 + eps[2]]).reshape(2, 1, 1)
    h = _dense_pair(x, agg1, sc1, wsel(W_a, 0, 2), bsel(b_a, 0, 2),
                    bsel(g, 0, 2), bsel(be, 0, 2), wsel(W_b, 0, 2),
                    bsel(b_b, 0, 2), act=True, shared_h=True)

    agg2 = _sc_agg_pair(h.reshape(2 * N, D), src_s2, dst3)
    sc2 = jnp.stack([1.0 + eps[1], 1.0 + eps[3]]).reshape(2, 1, 1)
    hout = _dense_pair(h, agg2, sc2, wsel(W_a, 1, 3), bsel(b_a, 1, 3),
                       bsel(g, 1, 3), bsel(be, 1, 3), wsel(W_b, 1, 3),
                       bsel(b_b, 1, 3), act=False, shared_h=False)

    ids_row = node_graph_ids.reshape(1, N)
    Wo1r = Wo1.reshape(2, D, D)
    Wo3p = jnp.pad(Wo3, ((0, 0), (0, D - 1)))
    bo3r = jnp.pad(bo3, (0, D - 1)).reshape(1, D)
    o = _readout(hout, ids_row, Wo1r, bo1.reshape(1, D), Wo2,
                 bo2.reshape(1, D), Wo3p, bo3r)
    return o[:, :1]


# speedup vs baseline: 7.6345x; 2.9417x over previous
import functools

import jax
import jax.numpy as jnp
from jax import lax
from jax.experimental import pallas as pl
from jax.experimental.pallas import tpu as pltpu
from jax.experimental.pallas import tpu_sc as plsc

N = 10000
D = 128
E = 320000
G = 128

NC = 2
NS = 16
CHUNK = 128
EPW = 20480
NCHUNKS = EPW // CHUNK
E_PAD = NS * EPW
NBUF = 2
IBLK = 8
NIB = 3
NBLK = NCHUNKS // IBLK
SH_ROWS = 10240
ZCOPIES = SH_ROWS // NS // CHUNK
OUT_PER_S = SH_ROWS // NS

HIGH = jax.lax.Precision.HIGHEST


def _sc_agg_pair(h2, src3, dst3, zrows):
    mesh = plsc.VectorSubcoreMesh(core_axis_name="c", subcore_axis_name="s")

    @functools.partial(
        pl.kernel,
        out_type=jax.ShapeDtypeStruct((NC, SH_ROWS, D), jnp.float32),
        mesh=mesh,
        scratch_types=[
            pltpu.VMEM((NIB, IBLK, CHUNK), jnp.int32),
            pltpu.VMEM((NIB, IBLK, CHUNK), jnp.int32),
            pltpu.VMEM((NBUF, CHUNK, D), jnp.float32),
            pltpu.VMEM_SHARED((SH_ROWS, D), jnp.float32),
            [pltpu.SemaphoreType.DMA] * NBUF,
            [pltpu.SemaphoreType.DMA] * NBUF,
            pltpu.SemaphoreType.DMA,
        ],
    )
    def agg_kernel(h_hbm, src_hbm, dst_hbm, z_hbm, out_hbm, sblk, dblk, rows,
                   shared, sg, ss, si):
        c = lax.axis_index("c")
        s = lax.axis_index("s")

        zbase = s * OUT_PER_S
        pltpu.sync_copy(z_hbm, shared.at[pl.ds(zbase, OUT_PER_S)])
        plsc.subcore_barrier()

        def idx_issue(kb, p):
            pltpu.async_copy(src_hbm.at[c, s, pl.ds(kb * IBLK, IBLK)],
                             sblk.at[p], si)
            pltpu.async_copy(dst_hbm.at[c, s, pl.ds(kb * IBLK, IBLK)],
                             dblk.at[p], si)

        def idx_wait():
            pltpu.make_async_copy(src_hbm.at[c, s, pl.ds(0, IBLK)],
                                  sblk.at[0], si).wait()
            pltpu.make_async_copy(dst_hbm.at[c, s, pl.ds(0, IBLK)],
                                  dblk.at[0], si).wait()

        def g_issue(p, pos, b):
            pltpu.async_copy(h_hbm.at[sblk.at[p, pos]], rows.at[b], sg[b])

        def g_wait(b):
            pltpu.make_async_copy(h_hbm.at[sblk.at[0, 0]], rows.at[b],
                                  sg[b]).wait()

        def s_issue(p, pos, b):
            pltpu.async_copy(rows.at[b], shared.at[dblk.at[p, pos]], ss[b],
                             add=True)

        def s_wait(b):
            pltpu.make_async_copy(rows.at[b], shared.at[dblk.at[0, 0]],
                                  ss[b]).wait()

        idx_issue(0, 0)

        def block(kb, carry):
            p = lax.rem(kb, NIB)
            pm1 = lax.rem(kb + (NIB - 1), NIB)
            pnx = lax.rem(kb + 1, NIB)
            idx_wait()

            @pl.when(kb < NBLK - 1)
            def _():
                idx_issue(kb + 1, pnx)

            for pos in range(IBLK):
                b = pos % NBUF
                if pos >= NBUF:
                    s_wait(b)
                else:
                    @pl.when(kb >= 1)
                    def _():
                        s_wait(b)
                g_issue(p, pos, b)
                if pos >= 1:
                    g_wait(1 - b)
                    s_issue(p, pos - 1, 1 - b)
                else:
                    @pl.when(kb >= 1)
                    def _():
                        g_wait((IBLK - 1) % NBUF)
                        s_issue(pm1, IBLK - 1, (IBLK - 1) % NBUF)
            return carry
        lax.fori_loop(0, NBLK, block, 0)

        lastb = (IBLK - 1) % NBUF
        g_wait(lastb)
        s_issue((NBLK - 1) % NIB, IBLK - 1, lastb)
        for b in range(NBUF):
            s_wait(b)
        plsc.subcore_barrier()

        obase = s * OUT_PER_S
        pltpu.sync_copy(shared.at[pl.ds(obase, OUT_PER_S)],
                        out_hbm.at[c, pl.ds(obase, OUT_PER_S)])

    return agg_kernel(h2, src3, dst3, zrows)


def _dense_pair(h, agg, scales, Wa2, ba2, g2, be2, Wb2, bb2, act, shared_h):
    def body(h_ref, a_ref, sc_ref, wa_ref, ba_ref, g_ref, be_ref, wb_ref,
             bb_ref, o_ref):
        hb = h_ref[...] if shared_h else h_ref[0]
        z = sc_ref[0] * hb + a_ref[0, :N]
        z = jnp.dot(z, wa_ref[0], precision=HIGH,
                    preferred_element_type=jnp.float32) + ba_ref[0]
        m = jnp.mean(z, axis=0, keepdims=True)
        v = jnp.mean(jnp.square(z - m), axis=0, keepdims=True)
        z = g_ref[0] * (z - m) / jnp.sqrt(v + 1e-5) + be_ref[0]
        z = jnp.maximum(z, 0.0)
        z = jnp.dot(z, wb_ref[0], precision=HIGH,
                    preferred_element_type=jnp.float32) + bb_ref[0]
        if act:
            z = jnp.maximum(z, 0.0)
        o_ref[0] = z

    h_spec = (pl.BlockSpec((N, D), lambda b: (0, 0)) if shared_h
              else pl.BlockSpec((1, N, D), lambda b: (b, 0, 0)))
    return pl.pallas_call(
        body,
        grid=(2,),
        in_specs=[
            h_spec,
            pl.BlockSpec((1, SH_ROWS, D), lambda b: (b, 0, 0)),
            pl.BlockSpec((1, 1, 1), lambda b: (b, 0, 0)),
            pl.BlockSpec((1, D, D), lambda b: (b, 0, 0)),
            pl.BlockSpec((1, 1, D), lambda b: (b, 0, 0)),
            pl.BlockSpec((1, 1, D), lambda b: (b, 0, 0)),
            pl.BlockSpec((1, 1, D), lambda b: (b, 0, 0)),
            pl.BlockSpec((1, D, D), lambda b: (b, 0, 0)),
            pl.BlockSpec((1, 1, D), lambda b: (b, 0, 0)),
        ],
        out_specs=pl.BlockSpec((1, N, D), lambda b: (b, 0, 0)),
        out_shape=jax.ShapeDtypeStruct((2, N, D), jnp.float32),
    )(h, agg, scales, Wa2, ba2, g2, be2, Wb2, bb2)


def _readout(hpair, ids_row, Wo1r, bo1r, Wo2, bo2r, Wo3p, bo3r):
    def body(h_ref, ids_ref, w1_ref, b1_ref, w2_ref, b2_ref, w3_ref, b3_ref,
             o_ref):
        gi = lax.broadcasted_iota(jnp.int32, (G, N), 0)
        S = jnp.where(gi == ids_ref[...], 1.0, 0.0)
        hg0 = jnp.dot(S, h_ref[0], precision=HIGH,
                      preferred_element_type=jnp.float32)
        hg1 = jnp.dot(S, h_ref[1], precision=HIGH,
                      preferred_element_type=jnp.float32)
        o = (jnp.dot(hg0, w1_ref[0], precision=HIGH,
                     preferred_element_type=jnp.float32)
             + jnp.dot(hg1, w1_ref[1], precision=HIGH,
                       preferred_element_type=jnp.float32)
             + b1_ref[...])
        o = jnp.maximum(o, 0.0)
        o = jnp.dot(o, w2_ref[...], precision=HIGH,
                    preferred_element_type=jnp.float32) + b2_ref[...]
        o = jnp.maximum(o, 0.0)
        o = jnp.dot(o, w3_ref[...], precision=HIGH,
                    preferred_element_type=jnp.float32) + b3_ref[...]
        o_ref[...] = o

    return pl.pallas_call(
        body,
        out_shape=jax.ShapeDtypeStruct((G, D), jnp.float32),
    )(hpair, ids_row, Wo1r, bo1r, Wo2, bo2r, Wo3p, bo3r)


def kernel(x, edge_index_0, edge_index_1, node_graph_ids, eps, W_a, b_a, g, be,
           W_b, b_b, Wo1, bo1, Wo2, bo2, Wo3, bo3):
    pad = E_PAD - E

    pad_src = jnp.arange(pad, dtype=jnp.int32) * 37 % N
    pad_dst = N + (jnp.arange(pad, dtype=jnp.int32) % (SH_ROWS - N))

    def prep(a, off, fill):
        return jnp.concatenate([a + off, fill]).reshape(NS, NCHUNKS, CHUNK)

    src_s1 = jnp.stack([prep(edge_index_0[0], 0, pad_src),
                        prep(edge_index_1[0], 0, pad_src)])
    src_s2 = jnp.stack([prep(edge_index_0[0], 0, pad_src),
                        prep(edge_index_1[0], N, pad_src)])
    dst3 = jnp.stack([prep(edge_index_0[1], 0, pad_dst),
                      prep(edge_index_1[1], 0, pad_dst)])

    def wsel(w, i0, i1):
        return jnp.stack([w[i0], w[i1]])

    def bsel(w, i0, i1):
        return jnp.stack([w[i0], w[i1]]).reshape(2, 1, D)

    zrows = jnp.zeros((OUT_PER_S, D), jnp.float32)
    agg1 = _sc_agg_pair(x, src_s1, dst3, zrows)
    sc1 = jnp.stack([1.0 + eps[0], 1.0 + eps[2]]).reshape(2, 1, 1)
    h = _dense_pair(x, agg1, sc1, wsel(W_a, 0, 2), bsel(b_a, 0, 2),
                    bsel(g, 0, 2), bsel(be, 0, 2), wsel(W_b, 0, 2),
                    bsel(b_b, 0, 2), act=True, shared_h=True)

    agg2 = _sc_agg_pair(h.reshape(2 * N, D), src_s2, dst3, zrows)
    sc2 = jnp.stack([1.0 + eps[1], 1.0 + eps[3]]).reshape(2, 1, 1)
    hout = _dense_pair(h, agg2, sc2, wsel(W_a, 1, 3), bsel(b_a, 1, 3),
                       bsel(g, 1, 3), bsel(be, 1, 3), wsel(W_b, 1, 3),
                       bsel(b_b, 1, 3), act=False, shared_h=False)

    ids_row = node_graph_ids.reshape(1, N)
    Wo1r = Wo1.reshape(2, D, D)
    Wo3p = jnp.pad(Wo3, ((0, 0), (0, D - 1)))
    bo3r = jnp.pad(bo3, (0, D - 1)).reshape(1, D)
    o = _readout(hout, ids_row, Wo1r, bo1.reshape(1, D), Wo2,
                 bo2.reshape(1, D), Wo3p, bo3r)
    return o[:, :1]
